# baseline (speedup 1.0000x reference)
import functools

import jax
import jax.numpy as jnp
from jax import lax
from jax.experimental import pallas as pl
from jax.experimental.pallas import tpu as pltpu
from jax.experimental.pallas import tpu_sc as plsc

BATCH = 16384
EMBED = 32
NC = 2
NS = 16
NW = NC * NS
BPW = BATCH // NW
LANES = 16


def _sc_body(idx_u_hbm, idx_m_hbm, uemb_hbm, memb_hbm, ub_hbm, mb_hbm,
             partials_hbm, bsum_hbm,
             idxu_v, idxm_v, urows_v, mrows_v, ubv, mbv, bsum_v, acc_v,
             sem0, sem1, sem2, sem3):
    wid = lax.axis_index("s") * NC + lax.axis_index("c")
    base = wid * BPW

    pltpu.sync_copy(idx_u_hbm.at[pl.ds(base, BPW)], idxu_v)
    pltpu.sync_copy(idx_m_hbm.at[pl.ds(base, BPW)], idxm_v)

    cu = pltpu.async_copy(uemb_hbm.at[idxu_v], urows_v, sem0)
    cm = pltpu.async_copy(memb_hbm.at[idxm_v], mrows_v, sem1)
    cub = pltpu.async_copy(ub_hbm.at[idxu_v], ubv, sem2)
    cmb = pltpu.async_copy(mb_hbm.at[idxm_v], mbv, sem3)
    cub.wait()
    cmb.wait()

    def bias_body(j, c):
        s = ubv[pl.ds(j * LANES, LANES)] + mbv[pl.ds(j * LANES, LANES)]
        bsum_v[pl.ds(j * LANES, LANES)] = s
        return c
    lax.fori_loop(0, BPW // LANES, bias_body, 0)
    pltpu.sync_copy(bsum_v, bsum_hbm.at[pl.ds(base, BPW)])

    cu.wait()
    cm.wait()

    def dot_body(r, acc):
        a0 = urows_v[r, pl.ds(0, LANES)] * mrows_v[r, pl.ds(0, LANES)]
        a1 = urows_v[r, pl.ds(LANES, LANES)] * mrows_v[r, pl.ds(LANES, LANES)]
        return acc + a0 + a1
    acc = lax.fori_loop(0, BPW, dot_body, jnp.zeros((LANES,), jnp.float32))

    acc_v[...] = acc
    pltpu.sync_copy(acc_v, partials_hbm.at[wid])


def _sc_stage(idx_u, idx_m, uemb, memb, ub_flat, mb_flat):
    mesh = plsc.VectorSubcoreMesh(core_axis_name="c", subcore_axis_name="s")
    return pl.kernel(
        _sc_body,
        out_type=(
            jax.ShapeDtypeStruct((NW, LANES), jnp.float32),
            jax.ShapeDtypeStruct((BATCH,), jnp.float32),
        ),
        mesh=mesh,
        scratch_types=[
            pltpu.VMEM((BPW,), jnp.int32),
            pltpu.VMEM((BPW,), jnp.int32),
            pltpu.VMEM((BPW, EMBED), jnp.float32),
            pltpu.VMEM((BPW, EMBED), jnp.float32),
            pltpu.VMEM((BPW,), jnp.float32),
            pltpu.VMEM((BPW,), jnp.float32),
            pltpu.VMEM((BPW,), jnp.float32),
            pltpu.VMEM((LANES,), jnp.float32),
            pltpu.SemaphoreType.DMA,
            pltpu.SemaphoreType.DMA,
            pltpu.SemaphoreType.DMA,
            pltpu.SemaphoreType.DMA,
        ],
        compiler_params=pltpu.CompilerParams(use_tc_tiling_on_sc=False),
    )(idx_u, idx_m, uemb, memb, ub_flat, mb_flat)


def _tc_body(p_ref, b_ref, o_ref):
    s = jnp.sum(p_ref[...])
    o_ref[...] = jax.nn.sigmoid(b_ref[...] + s)


def _tc_finish(partials, bsum2d):
    return pl.pallas_call(
        _tc_body,
        out_shape=jax.ShapeDtypeStruct(bsum2d.shape, jnp.float32),
    )(partials, bsum2d)


@jax.jit
def kernel(inputs, user_embedding, user_bias, movie_embedding, movie_bias):
    idx_u = inputs[:, 0]
    idx_m = inputs[:, 1]
    partials, bsum = _sc_stage(
        idx_u, idx_m, user_embedding, movie_embedding,
        user_bias.reshape(-1), movie_bias.reshape(-1))
    out = _tc_finish(partials, bsum.reshape(128, 128))
    return out.reshape(BATCH, 1)
